# baseline (speedup 1.0000x reference)
import functools

import jax
import jax.numpy as jnp
from jax import lax
from jax.experimental import pallas as pl
from jax.experimental.pallas import tpu as pltpu
from jax.experimental.pallas import tpu_sc as plsc

_F = 128
_K = 65536
_NC = 2
_NS = 16
_NW = _NC * _NS
_CPW = _K // _NW
_C = 256
_NCHUNK = _CPW // _C

_mesh = plsc.VectorSubcoreMesh(core_axis_name="c", subcore_axis_name="s")


@functools.partial(
    pl.kernel,
    out_type=jax.ShapeDtypeStruct((_K, _F), jnp.float32),
    mesh=_mesh,
    scratch_types=[
        pltpu.VMEM((_F, _C), jnp.float32),
        pltpu.VMEM((_C, _F), jnp.float32),
    ],
    compiler_params=pltpu.CompilerParams(needs_layout_passes=False),
)
def _sc_transpose(q_hbm, out_hbm, in_v, out_v):
    wid = lax.axis_index("s") * _NC + lax.axis_index("c")
    col0 = wid * _CPW
    iota = lax.iota(jnp.int32, 16)
    rows = [f0 + iota for f0 in range(0, _F, 16)]
    for ch in range(_NCHUNK):
        c0 = col0 + ch * _C
        pltpu.sync_copy(q_hbm.at[:, pl.ds(c0, _C)], in_v)

        @plsc.parallel_loop(0, _C, unroll=4)
        def _row(k):
            cols = jnp.full((16,), k, jnp.int32)
            for j in range(_F // 16):
                v = plsc.load_gather(in_v, [rows[j], cols])
                out_v[k, pl.ds(j * 16, 16)] = v

        pltpu.sync_copy(out_v, out_hbm.at[pl.ds(c0, _C), :])


def kernel(queue):
    return _sc_transpose(queue)


# speedup vs baseline: 1.8550x; 1.8550x over previous
import functools

import jax
import jax.numpy as jnp
import numpy as np
from jax import lax
from jax.experimental import pallas as pl
from jax.experimental.pallas import tpu as pltpu
from jax.experimental.pallas import tpu_sc as plsc

_F = 128
_K = 65536
_NC = 2
_NS = 16
_NW = _NC * _NS
_CPW = _K // _NW
_C = 256
_NCHUNK = _CPW // _C

_mesh = plsc.VectorSubcoreMesh(core_axis_name="c", subcore_axis_name="s")


@functools.partial(
    pl.kernel,
    out_type=jax.ShapeDtypeStruct((_K, _F), jnp.float32),
    mesh=_mesh,
    scratch_types=[
        pltpu.VMEM((_F, _C), jnp.float32),
        pltpu.VMEM((_C, _F), jnp.float32),
    ],
    compiler_params=pltpu.CompilerParams(needs_layout_passes=False),
)
def _sc_transpose(q_hbm, out_hbm, in_v, out_v):
    wid = lax.axis_index("s") * _NC + lax.axis_index("c")
    col0 = wid * _CPW
    iota = lax.iota(jnp.int32, 16)
    frows = [iota + f0 for f0 in range(0, _F, 16)]
    coloffs = [jnp.bitwise_and(iota + d, 15) for d in range(16)]
    def _chunk(ch, carry):
        c0 = col0 + ch * _C
        pltpu.sync_copy(q_hbm.at[:, pl.ds(c0, _C)], in_v)

        @plsc.parallel_loop(0, _C // 16)
        def _tile(t):
            k0 = t * 16
            for d in range(16):
                kcols = k0 + coloffs[d]
                for j in range(_F // 16):
                    v = plsc.load_gather(in_v, [frows[j], kcols])
                    plsc.store_scatter(out_v, [kcols, frows[j]], v)

        pltpu.sync_copy(out_v, out_hbm.at[pl.ds(c0, _C), :])
        return carry

    lax.fori_loop(0, _NCHUNK, _chunk, 0)


def kernel(queue):
    return _sc_transpose(queue)
